# baseline (speedup 1.0000x reference)
import functools

import jax
import jax.numpy as jnp
from jax.experimental import pallas as pl
from jax.experimental.pallas import tpu as pltpu


def _fused_cell(x_ref, p_ref, o_ref, *, in_ch, cat_ch, out_ch):
    xb = x_ref[...]
    r0, r1, r2, r3 = 0, cat_ch, 2 * cat_ch, 3 * cat_ch
    rb = 4 * cat_ch
    wz = 0.5 * (p_ref[r0:r0 + in_ch, :] + p_ref[r1:r1 + in_ch, :])
    wh = p_ref[r2:r2 + in_ch, :] + p_ref[r3:r3 + in_ch, :]
    w = jnp.concatenate([wz, wh], axis=1)
    bcat = jnp.concatenate(
        [0.5 * p_ref[rb:rb + 1, :], p_ref[rb + 1:rb + 2, :]], axis=1)
    y = jnp.dot(xb, w, preferred_element_type=jnp.float32) + bcat
    g = jnp.tanh(y)
    h = jnp.maximum((1.0 - g[:, :out_ch]) * g[:, out_ch:], 0.0)
    r = jax.lax.dot_general(p_ref[rb + 2:rb + 3, :], h, (((1,), (1,)), ((), ())),
                            preferred_element_type=jnp.float32)
    o_ref[...] = 0.5 * r[0] + p_ref[rb + 3, 0]


def kernel(x, edge_index, edge_weight, W_z, b_z, W_r, b_r, W_h, b_h,
           W_lin, b_lin):
    del edge_index, edge_weight, W_r, b_r
    n, in_ch = x.shape
    cat_ch, out_ch = W_z.shape[-2:]

    packed = jnp.concatenate([
        W_z.reshape(2 * cat_ch, out_ch),
        W_h.reshape(2 * cat_ch, out_ch),
        b_z[None, :], b_h[None, :],
        W_lin.reshape(1, out_ch),
        jnp.concatenate([b_lin, jnp.zeros((out_ch - 1,), x.dtype)])[None, :],
    ], axis=0)

    block = 5120
    grid = (n + block - 1) // block

    out1d = pl.pallas_call(
        functools.partial(_fused_cell, in_ch=in_ch, cat_ch=cat_ch,
                          out_ch=out_ch),
        grid=(grid,),
        in_specs=[
            pl.BlockSpec((block, in_ch), lambda i: (i, 0)),
            pl.BlockSpec((4 * cat_ch + 4, out_ch), lambda i: (0, 0)),
        ],
        out_specs=pl.BlockSpec((block,), lambda i: (i,)),
        out_shape=jax.ShapeDtypeStruct((grid * block,), x.dtype),
        compiler_params=pltpu.CompilerParams(
            dimension_semantics=("parallel",)),
    )(x, packed)
    return out1d[:n, None]
